# baseline (speedup 1.0000x reference)
import jax
import jax.numpy as jnp
from jax import lax
from jax.experimental import pallas as pl
from jax.experimental.pallas import tpu as pltpu
from jax.experimental.pallas import tpu_sc as plsc

N_FIELDS = 26
VOCAB = 100000
DIM = 10
DIMP = 16
BATCH = 16384

NC = 2
NS = 16
L = 16
NW = NC * NS

ROWS_PER_W = BATCH * N_FIELDS // NW
G = 128
NG = ROWS_PER_W // G
VECS_PER_ROW = G // L

FIELD_STEP = (L % N_FIELDS) * VOCAB
FIELD_WRAP = N_FIELDS * VOCAB


def _body(idx_hbm, tab_hbm, out_hbm, idx_v, rows_v, gsem0, gsem1):
    wid = lax.axis_index("s") * NC + lax.axis_index("c")
    row0 = wid * ROWS_PER_W

    pltpu.sync_copy(idx_hbm.at[pl.ds(wid * NG, NG)], idx_v)

    foff0 = jax.lax.iota(jnp.int32, L) * VOCAB

    def idx_row(j, foff):
        for t in range(VECS_PER_ROW):
            sl = pl.ds(t * L, L)
            idx_v[j, sl] = idx_v[j, sl] + foff
            nxt = foff + FIELD_STEP
            foff = jnp.where(nxt >= FIELD_WRAP, nxt - FIELD_WRAP, nxt)
        return foff

    lax.fori_loop(0, NG, idx_row, foff0)

    gsems = (gsem0, gsem1)

    def gather(g, slot):
        return pltpu.make_async_copy(
            tab_hbm.at[idx_v.at[g]], rows_v.at[slot], gsems[slot])

    def copy_out(g, slot):
        pltpu.sync_copy(rows_v.at[slot],
                        out_hbm.at[pl.ds(row0 + g * G, G)])

    gather(0, 0).start()
    gather(1, 1).start()

    def step(j, _):
        g = 2 * j
        gather(g, 0).wait()
        copy_out(g, 0)
        gather(g + 2, 0).start()
        gather(g + 1, 1).wait()
        copy_out(g + 1, 1)
        gather(g + 3, 1).start()
        return 0

    lax.fori_loop(0, NG // 2 - 1, step, 0)
    gather(NG - 2, 0).wait()
    copy_out(NG - 2, 0)
    gather(NG - 1, 1).wait()
    copy_out(NG - 1, 1)


@jax.jit
def _group_embed(idx_flat, tab_pad):
    run = pl.kernel(
        _body,
        out_type=jax.ShapeDtypeStruct((BATCH * N_FIELDS, DIMP), jnp.float32),
        mesh=plsc.VectorSubcoreMesh(
            core_axis_name="c", subcore_axis_name="s",
            num_cores=NC, num_subcores=NS,
        ),
        scratch_types=[
            pltpu.VMEM((NG, G), jnp.int32),
            pltpu.VMEM((2, G, DIMP), jnp.float32),
            pltpu.SemaphoreType.DMA,
            pltpu.SemaphoreType.DMA,
        ],
        compiler_params=pltpu.CompilerParams(use_tc_tiling_on_sc=False),
    )
    return run(idx_flat, tab_pad)


def kernel(indices, tables):
    idx_flat = indices.reshape(BATCH * N_FIELDS // G, G)
    tab_pad = jnp.pad(tables.reshape(N_FIELDS * VOCAB, DIM),
                      ((0, 0), (0, DIMP - DIM)))
    out = _group_embed(idx_flat, tab_pad)
    return out[:, :DIM].reshape(BATCH, N_FIELDS * DIM)


# speedup vs baseline: 1.4606x; 1.4606x over previous
import jax
import jax.numpy as jnp
from jax import lax
from jax.experimental import pallas as pl
from jax.experimental.pallas import tpu as pltpu
from jax.experimental.pallas import tpu_sc as plsc

N_FIELDS = 26
VOCAB = 100000
DIM = 10
DIMP = 16
BATCH = 16384

NC = 2
NS = 16
L = 16
NW = NC * NS

ROWS_PER_W = BATCH * N_FIELDS // NW
G = 128
NG = ROWS_PER_W // G
VECS_PER_ROW = G // L

TAB_ROWS = N_FIELDS * VOCAB * DIM // DIMP
SLAB = N_FIELDS * VOCAB // DIMP

FIELD_STEP = (L % N_FIELDS) * VOCAB
FIELD_WRAP = N_FIELDS * VOCAB


def _body(idx_hbm, tab_hbm, out_hbm, idx_v, q_v, rows_v, out_v, gsem0,
          gsem1):
    wid = lax.axis_index("s") * NC + lax.axis_index("c")
    row0 = wid * ROWS_PER_W

    pltpu.sync_copy(idx_hbm.at[pl.ds(wid * NG, NG)], idx_v)

    foff0 = jax.lax.iota(jnp.int32, L) * VOCAB

    def idx_row(j, foff):
        for t in range(VECS_PER_ROW):
            sl = pl.ds(t * L, L)
            idx_v[j, sl] = idx_v[j, sl] + foff
            nxt = foff + FIELD_STEP
            foff = jnp.where(nxt >= FIELD_WRAP, nxt - FIELD_WRAP, nxt)
        return foff

    lax.fori_loop(0, NG, idx_row, foff0)

    gsems = (gsem0, gsem1)
    iota = jax.lax.iota(jnp.int32, L)

    def build_q(g, slot):
        for t in range(VECS_PER_ROW):
            sl = pl.ds(t * L, L)
            hi = jax.lax.shift_right_logical(idx_v[g, sl], 4)
            for k in range(DIM):
                q_v[slot, k, sl] = hi + (k * SLAB)

    def fire(g, slot):
        for k in range(DIM):
            pltpu.make_async_copy(
                tab_hbm.at[q_v.at[slot, k]],
                rows_v.at[slot, k],
                gsems[slot],
            ).start()

    def drain(slot):
        for k in range(DIM):
            pltpu.make_async_copy(
                tab_hbm.at[q_v.at[slot, k]],
                rows_v.at[slot, k],
                gsems[slot],
            ).wait()

    def reassemble(g, slot):
        for t in range(VECS_PER_ROW):
            sl = pl.ds(t * L, L)
            lo = jax.lax.bitwise_and(idx_v[g, sl], 15)
            jv = iota + (t * L)
            for k in range(DIM):
                kv = jnp.full((L,), k, jnp.int32)
                vals = plsc.load_gather(rows_v.at[slot], [kv, jv, lo])
                plsc.store_scatter(out_v.at[slot], [jv, kv], vals)

    def copy_out(g, slot):
        pltpu.sync_copy(out_v.at[slot],
                        out_hbm.at[pl.ds(row0 + g * G, G)])

    build_q(0, 0)
    fire(0, 0)
    build_q(1, 1)
    fire(1, 1)

    def step(j, _):
        g = 2 * j
        drain(0)
        reassemble(g, 0)
        copy_out(g, 0)
        build_q(g + 2, 0)
        fire(g + 2, 0)
        drain(1)
        reassemble(g + 1, 1)
        copy_out(g + 1, 1)
        build_q(g + 3, 1)
        fire(g + 3, 1)
        return 0

    lax.fori_loop(0, NG // 2 - 1, step, 0)
    drain(0)
    reassemble(NG - 2, 0)
    copy_out(NG - 2, 0)
    drain(1)
    reassemble(NG - 1, 1)
    copy_out(NG - 1, 1)


@jax.jit
def _group_embed(idx_flat, tab_kmaj):
    run = pl.kernel(
        _body,
        out_type=jax.ShapeDtypeStruct((BATCH * N_FIELDS, DIMP), jnp.float32),
        mesh=plsc.VectorSubcoreMesh(
            core_axis_name="c", subcore_axis_name="s",
            num_cores=NC, num_subcores=NS,
        ),
        scratch_types=[
            pltpu.VMEM((NG, G), jnp.int32),
            pltpu.VMEM((2, DIM, G), jnp.int32),
            pltpu.VMEM((2, DIM, G, DIMP), jnp.float32),
            pltpu.VMEM((2, G, DIMP), jnp.float32),
            pltpu.SemaphoreType.DMA,
            pltpu.SemaphoreType.DMA,
        ],
        compiler_params=pltpu.CompilerParams(
            use_tc_tiling_on_sc=False, needs_layout_passes=False),
    )
    return run(idx_flat, tab_kmaj)


def kernel(indices, tables):
    idx_flat = indices.reshape(BATCH * N_FIELDS // G, G)
    tab_kmaj = tables.transpose(2, 0, 1).reshape(TAB_ROWS, DIMP)
    out = _group_embed(idx_flat, tab_kmaj)
    return out[:, :DIM].reshape(BATCH, N_FIELDS * DIM)


# speedup vs baseline: 4.3544x; 2.9812x over previous
import jax
import jax.numpy as jnp
from jax import lax
from jax.experimental import pallas as pl
from jax.experimental.pallas import tpu as pltpu
from jax.experimental.pallas import tpu_sc as plsc

N_FIELDS = 26
VOCAB = 100000
DIM = 10
DIMP = 16
BATCH = 16384

NC = 2
NS = 16
L = 16
NW = NC * NS

ROWS_PER_W = BATCH * N_FIELDS // NW
G = 128
NG = ROWS_PER_W // G
VECS_PER_ROW = G // L

TAB_ROWS = N_FIELDS * VOCAB * DIM // DIMP
SLAB = N_FIELDS * VOCAB // DIMP

FIELD_STEP = (L % N_FIELDS) * VOCAB
FIELD_WRAP = N_FIELDS * VOCAB


def _body(idx_hbm, tab_hbm, out_hbm, idx_v, q_v, rows_v, out_v, gsem0,
          gsem1):
    wid = lax.axis_index("s") * NC + lax.axis_index("c")
    row0 = wid * ROWS_PER_W

    pltpu.sync_copy(idx_hbm.at[pl.ds(wid * NG, NG)], idx_v)

    foff0 = jax.lax.iota(jnp.int32, L) * VOCAB

    def idx_row(j, foff):
        for t in range(VECS_PER_ROW):
            sl = pl.ds(t * L, L)
            idx_v[j, sl] = idx_v[j, sl] + foff
            nxt = foff + FIELD_STEP
            foff = jnp.where(nxt >= FIELD_WRAP, nxt - FIELD_WRAP, nxt)
        return foff

    lax.fori_loop(0, NG, idx_row, foff0)

    gsems = (gsem0, gsem1)
    iota = jax.lax.iota(jnp.int32, L)

    def build_q(g, slot):
        for t in range(VECS_PER_ROW):
            sl = pl.ds(t * L, L)
            hi = jax.lax.shift_right_logical(idx_v[g, sl], 4)
            for k in range(DIM):
                q_v[slot, k, sl] = hi + (k * SLAB)

    def fire(g, slot):
        for k in range(DIM):
            pltpu.make_async_copy(
                tab_hbm.at[q_v.at[slot, k]],
                rows_v.at[slot, k],
                gsems[slot],
            ).start()

    def drain(slot):
        for k in range(DIM):
            pltpu.make_async_copy(
                tab_hbm.at[q_v.at[slot, k]],
                rows_v.at[slot, k],
                gsems[slot],
            ).wait()

    def reassemble(g, slot):
        for t in range(VECS_PER_ROW):
            sl = pl.ds(t * L, L)
            lo = jax.lax.bitwise_and(idx_v[g, sl], 15)
            jv = iota + (t * L)
            for k in range(DIM):
                kv = jnp.full((L,), k, jnp.int32)
                vals = plsc.load_gather(rows_v.at[slot], [kv, jv, lo])
                plsc.store_scatter(out_v.at[slot], [jv, kv], vals)

    def copy_out(g, slot):
        pltpu.sync_copy(out_v.at[slot],
                        out_hbm.at[pl.ds(row0 + g * G, G)])

    build_q(0, 0)
    fire(0, 0)
    build_q(1, 1)
    fire(1, 1)

    def step(j, _):
        g = 2 * j
        drain(0)
        reassemble(g, 0)
        copy_out(g, 0)
        build_q(g + 2, 0)
        fire(g + 2, 0)
        drain(1)
        reassemble(g + 1, 1)
        copy_out(g + 1, 1)
        build_q(g + 3, 1)
        fire(g + 3, 1)
        return 0

    lax.fori_loop(0, NG // 2 - 1, step, 0)
    drain(0)
    reassemble(NG - 2, 0)
    copy_out(NG - 2, 0)
    drain(1)
    reassemble(NG - 1, 1)
    copy_out(NG - 1, 1)


def _detile_body(tab_hbm, out_hbm, buf_v, ksem):
    wid = lax.axis_index("s") * NC + lax.axis_index("c")
    nlines = jnp.where(wid < 4, 9, 8)
    base = jnp.where(wid < 4, wid * 9, 36 + (wid - 4) * 8)

    def line(i, _):
        u = base + i
        k = u // N_FIELDS
        f = u % N_FIELDS
        pltpu.make_async_copy(tab_hbm.at[k, f], buf_v, ksem).start()
        pltpu.make_async_copy(tab_hbm.at[k, f], buf_v, ksem).wait()
        pltpu.sync_copy(
            buf_v,
            out_hbm.at[pl.ds(k * (N_FIELDS * VOCAB) + f * VOCAB, VOCAB)])
        return 0

    lax.fori_loop(0, nlines, line, 0)


@jax.jit
def _detile(tab_t):
    run = pl.kernel(
        _detile_body,
        out_type=jax.ShapeDtypeStruct((N_FIELDS * VOCAB * DIM,),
                                      jnp.float32),
        mesh=plsc.VectorSubcoreMesh(
            core_axis_name="c", subcore_axis_name="s",
            num_cores=NC, num_subcores=NS,
        ),
        scratch_types=[
            pltpu.VMEM((VOCAB,), jnp.float32),
            pltpu.SemaphoreType.DMA,
        ],
        compiler_params=pltpu.CompilerParams(use_tc_tiling_on_sc=True),
    )
    return run(tab_t)


@jax.jit
def _group_embed(idx_flat, tab_kmaj):
    run = pl.kernel(
        _body,
        out_type=jax.ShapeDtypeStruct((BATCH * N_FIELDS, DIMP), jnp.float32),
        mesh=plsc.VectorSubcoreMesh(
            core_axis_name="c", subcore_axis_name="s",
            num_cores=NC, num_subcores=NS,
        ),
        scratch_types=[
            pltpu.VMEM((NG, G), jnp.int32),
            pltpu.VMEM((2, DIM, G), jnp.int32),
            pltpu.VMEM((2, DIM, G, DIMP), jnp.float32),
            pltpu.VMEM((2, G, DIMP), jnp.float32),
            pltpu.SemaphoreType.DMA,
            pltpu.SemaphoreType.DMA,
        ],
        compiler_params=pltpu.CompilerParams(
            use_tc_tiling_on_sc=False, needs_layout_passes=False),
    )
    return run(idx_flat, tab_kmaj)


def kernel(indices, tables):
    idx_flat = indices.reshape(BATCH * N_FIELDS // G, G)
    tab_kmaj = _detile(tables.transpose(2, 0, 1)).reshape(TAB_ROWS, DIMP)
    out = _group_embed(idx_flat, tab_kmaj)
    return out[:, :DIM].reshape(BATCH, N_FIELDS * DIM)


# speedup vs baseline: 4.4442x; 1.0206x over previous
import jax
import jax.numpy as jnp
from jax import lax
from jax.experimental import pallas as pl
from jax.experimental.pallas import tpu as pltpu
from jax.experimental.pallas import tpu_sc as plsc

N_FIELDS = 26
VOCAB = 100000
DIM = 10
DIMP = 16
BATCH = 16384

NC = 2
NS = 16
L = 16
NW = NC * NS

ROWS_PER_W = BATCH * N_FIELDS // NW
G = 128
NG = ROWS_PER_W // G
VECS_PER_ROW = G // L

TAB_ROWS = N_FIELDS * VOCAB * DIM // DIMP
SLAB = N_FIELDS * VOCAB // DIMP

FIELD_STEP = (L % N_FIELDS) * VOCAB
FIELD_WRAP = N_FIELDS * VOCAB


def _body(idx_hbm, tab_hbm, out_hbm, idx_v, q_v, rows_v, out_v, gsem0,
          gsem1, gsem2, gsem3):
    wid = lax.axis_index("s") * NC + lax.axis_index("c")
    row0 = wid * ROWS_PER_W

    pltpu.sync_copy(idx_hbm.at[pl.ds(wid * NG, NG)], idx_v)

    foff0 = jax.lax.iota(jnp.int32, L) * VOCAB

    def idx_row(j, foff):
        for t in range(VECS_PER_ROW):
            sl = pl.ds(t * L, L)
            idx_v[j, sl] = idx_v[j, sl] + foff
            nxt = foff + FIELD_STEP
            foff = jnp.where(nxt >= FIELD_WRAP, nxt - FIELD_WRAP, nxt)
        return foff

    lax.fori_loop(0, NG, idx_row, foff0)

    gsems = (gsem0, gsem1, gsem2, gsem3)
    iota = jax.lax.iota(jnp.int32, L)

    def build_q(g, slot):
        for t in range(VECS_PER_ROW):
            sl = pl.ds(t * L, L)
            hi = jax.lax.shift_right_logical(idx_v[g, sl], 4)
            for k in range(DIM):
                q_v[slot, k, sl] = hi + (k * SLAB)

    def fire(g, slot):
        for k in range(DIM):
            pltpu.make_async_copy(
                tab_hbm.at[q_v.at[slot, k]],
                rows_v.at[slot, k],
                gsems[slot],
            ).start()

    def drain(slot):
        for k in range(DIM):
            pltpu.make_async_copy(
                tab_hbm.at[q_v.at[slot, k]],
                rows_v.at[slot, k],
                gsems[slot],
            ).wait()

    def reassemble(g, slot):
        for t in range(VECS_PER_ROW):
            sl = pl.ds(t * L, L)
            lo = jax.lax.bitwise_and(idx_v[g, sl], 15)
            jv = iota + (t * L)
            for k in range(DIM):
                kv = jnp.full((L,), k, jnp.int32)
                vals = plsc.load_gather(rows_v.at[slot], [kv, jv, lo])
                plsc.store_scatter(out_v.at[slot], [jv, kv], vals)

    def copy_out(g, slot):
        pltpu.sync_copy(out_v.at[slot],
                        out_hbm.at[pl.ds(row0 + g * G, G)])

    NSLOT = 4
    for s in range(NSLOT):
        build_q(s, s)
        fire(s, s)

    def step(j, _):
        g0 = NSLOT * j
        for s in range(NSLOT):
            g = g0 + s
            drain(s)
            reassemble(g, s)
            copy_out(g, s)
            build_q(g + NSLOT, s)
            fire(g + NSLOT, s)
        return 0

    lax.fori_loop(0, NG // NSLOT - 1, step, 0)
    for s in range(NSLOT):
        g = NG - NSLOT + s
        drain(s)
        reassemble(g, s)
        copy_out(g, s)


def _detile_body(tab_hbm, out_hbm, buf_v, ksem):
    wid = lax.axis_index("s") * NC + lax.axis_index("c")
    nlines = jnp.where(wid < 4, 9, 8)
    base = jnp.where(wid < 4, wid * 9, 36 + (wid - 4) * 8)

    def line(i, _):
        u = base + i
        k = u // N_FIELDS
        f = u % N_FIELDS
        pltpu.make_async_copy(tab_hbm.at[k, f], buf_v, ksem).start()
        pltpu.make_async_copy(tab_hbm.at[k, f], buf_v, ksem).wait()
        pltpu.sync_copy(
            buf_v,
            out_hbm.at[pl.ds(k * (N_FIELDS * VOCAB) + f * VOCAB, VOCAB)])
        return 0

    lax.fori_loop(0, nlines, line, 0)


@jax.jit
def _detile(tab_t):
    run = pl.kernel(
        _detile_body,
        out_type=jax.ShapeDtypeStruct((N_FIELDS * VOCAB * DIM,),
                                      jnp.float32),
        mesh=plsc.VectorSubcoreMesh(
            core_axis_name="c", subcore_axis_name="s",
            num_cores=NC, num_subcores=NS,
        ),
        scratch_types=[
            pltpu.VMEM((VOCAB,), jnp.float32),
            pltpu.SemaphoreType.DMA,
        ],
        compiler_params=pltpu.CompilerParams(use_tc_tiling_on_sc=True),
    )
    return run(tab_t)


@jax.jit
def _group_embed(idx_flat, tab_kmaj):
    run = pl.kernel(
        _body,
        out_type=jax.ShapeDtypeStruct((BATCH * N_FIELDS, DIMP), jnp.float32),
        mesh=plsc.VectorSubcoreMesh(
            core_axis_name="c", subcore_axis_name="s",
            num_cores=NC, num_subcores=NS,
        ),
        scratch_types=[
            pltpu.VMEM((NG, G), jnp.int32),
            pltpu.VMEM((4, DIM, G), jnp.int32),
            pltpu.VMEM((4, DIM, G, DIMP), jnp.float32),
            pltpu.VMEM((4, G, DIMP), jnp.float32),
            pltpu.SemaphoreType.DMA,
            pltpu.SemaphoreType.DMA,
            pltpu.SemaphoreType.DMA,
            pltpu.SemaphoreType.DMA,
        ],
        compiler_params=pltpu.CompilerParams(
            use_tc_tiling_on_sc=False, needs_layout_passes=False),
    )
    return run(idx_flat, tab_kmaj)


def kernel(indices, tables):
    idx_flat = indices.reshape(BATCH * N_FIELDS // G, G)
    tab_kmaj = _detile(tables.transpose(2, 0, 1)).reshape(TAB_ROWS, DIMP)
    out = _group_embed(idx_flat, tab_kmaj)
    return out[:, :DIM].reshape(BATCH, N_FIELDS * DIM)


# speedup vs baseline: 7.6205x; 1.7147x over previous
import jax
import jax.numpy as jnp
from jax import lax
from jax.experimental import pallas as pl
from jax.experimental.pallas import tpu as pltpu
from jax.experimental.pallas import tpu_sc as plsc

N_FIELDS = 26
VOCAB = 100000
DIM = 10
DIMP = 16
BATCH = 16384

NC = 2
NS = 16
L = 16
NW = NC * NS

ROWS_PER_W = BATCH * N_FIELDS // NW
G = 128
NG = ROWS_PER_W // G
VECS_PER_ROW = G // L

TAB_ROWS = N_FIELDS * VOCAB * DIM // DIMP
SLAB = N_FIELDS * VOCAB // DIMP

FIELD_STEP = (L % N_FIELDS) * VOCAB
FIELD_WRAP = N_FIELDS * VOCAB


def _body(idx_hbm, tab_hbm, out_hbm, idx_v, q_v, rows_v, out_v, gsem0,
          gsem1, gsem2, gsem3):
    wid = lax.axis_index("s") * NC + lax.axis_index("c")
    row0 = wid * ROWS_PER_W

    pltpu.sync_copy(idx_hbm.at[pl.ds(wid * NG, NG)], idx_v)

    foff0 = jax.lax.iota(jnp.int32, L) * VOCAB

    def idx_row(j, foff):
        for t in range(VECS_PER_ROW):
            sl = pl.ds(t * L, L)
            idx_v[j, sl] = idx_v[j, sl] + foff
            nxt = foff + FIELD_STEP
            foff = jnp.where(nxt >= FIELD_WRAP, nxt - FIELD_WRAP, nxt)
        return foff

    lax.fori_loop(0, NG, idx_row, foff0)

    gsems = (gsem0, gsem1, gsem2, gsem3)
    iota = jax.lax.iota(jnp.int32, L)

    def build_q(g, slot):
        for t in range(VECS_PER_ROW):
            sl = pl.ds(t * L, L)
            hi = jax.lax.shift_right_logical(idx_v[g, sl], 4)
            for k in range(DIM):
                q_v[slot, k, sl] = hi + (k * SLAB)

    def fire(g, slot):
        for k in range(DIM):
            pltpu.make_async_copy(
                tab_hbm.at[q_v.at[slot, k]],
                rows_v.at[slot, k],
                gsems[slot],
            ).start()

    def drain(slot):
        for k in range(DIM):
            pltpu.make_async_copy(
                tab_hbm.at[q_v.at[slot, k]],
                rows_v.at[slot, k],
                gsems[slot],
            ).wait()

    def reassemble(g, slot):
        for t in range(VECS_PER_ROW):
            sl = pl.ds(t * L, L)
            lo = jax.lax.bitwise_and(idx_v[g, sl], 15)
            jv = iota + (t * L)
            jd = jv * DIM
            for k in range(DIM):
                kv = jnp.full((L,), k, jnp.int32)
                vals = plsc.load_gather(rows_v.at[slot], [kv, jv, lo])
                plsc.store_scatter(out_v.at[slot], [jd + k], vals)

    def copy_out(g, slot):
        pltpu.sync_copy(out_v.at[slot],
                        out_hbm.at[pl.ds((row0 + g * G) * DIM, G * DIM)])

    NSLOT = 4
    for s in range(NSLOT):
        build_q(s, s)
        fire(s, s)

    def step(j, _):
        g0 = NSLOT * j
        for s in range(NSLOT):
            g = g0 + s
            drain(s)
            reassemble(g, s)
            copy_out(g, s)
            build_q(g + NSLOT, s)
            fire(g + NSLOT, s)
        return 0

    lax.fori_loop(0, NG // NSLOT - 1, step, 0)
    for s in range(NSLOT):
        g = NG - NSLOT + s
        drain(s)
        reassemble(g, s)
        copy_out(g, s)


def _detile_body(tab_hbm, out_hbm, buf_v, ksem):
    wid = lax.axis_index("s") * NC + lax.axis_index("c")
    nlines = jnp.where(wid < 4, 9, 8)
    base = jnp.where(wid < 4, wid * 9, 36 + (wid - 4) * 8)

    def line(i, _):
        u = base + i
        k = u // N_FIELDS
        f = u % N_FIELDS
        pltpu.make_async_copy(tab_hbm.at[k, f], buf_v, ksem).start()
        pltpu.make_async_copy(tab_hbm.at[k, f], buf_v, ksem).wait()
        pltpu.sync_copy(
            buf_v,
            out_hbm.at[pl.ds(k * (N_FIELDS * VOCAB) + f * VOCAB, VOCAB)])
        return 0

    lax.fori_loop(0, nlines, line, 0)


@jax.jit
def _detile(tab_t):
    run = pl.kernel(
        _detile_body,
        out_type=jax.ShapeDtypeStruct((N_FIELDS * VOCAB * DIM,),
                                      jnp.float32),
        mesh=plsc.VectorSubcoreMesh(
            core_axis_name="c", subcore_axis_name="s",
            num_cores=NC, num_subcores=NS,
        ),
        scratch_types=[
            pltpu.VMEM((VOCAB,), jnp.float32),
            pltpu.SemaphoreType.DMA,
        ],
        compiler_params=pltpu.CompilerParams(use_tc_tiling_on_sc=True),
    )
    return run(tab_t)


@jax.jit
def _group_embed(idx_flat, tab_kmaj):
    run = pl.kernel(
        _body,
        out_type=jax.ShapeDtypeStruct((BATCH * N_FIELDS * DIM,),
                                      jnp.float32),
        mesh=plsc.VectorSubcoreMesh(
            core_axis_name="c", subcore_axis_name="s",
            num_cores=NC, num_subcores=NS,
        ),
        scratch_types=[
            pltpu.VMEM((NG, G), jnp.int32),
            pltpu.VMEM((4, DIM, G), jnp.int32),
            pltpu.VMEM((4, DIM, G, DIMP), jnp.float32),
            pltpu.VMEM((4, G * DIM), jnp.float32),
            pltpu.SemaphoreType.DMA,
            pltpu.SemaphoreType.DMA,
            pltpu.SemaphoreType.DMA,
            pltpu.SemaphoreType.DMA,
        ],
        compiler_params=pltpu.CompilerParams(
            use_tc_tiling_on_sc=False, needs_layout_passes=False),
    )
    return run(idx_flat, tab_kmaj)


def kernel(indices, tables):
    idx_flat = indices.reshape(BATCH * N_FIELDS // G, G)
    tab_kmaj = _detile(tables.transpose(2, 0, 1)).reshape(TAB_ROWS, DIMP)
    out = _group_embed(idx_flat, tab_kmaj)
    return out.reshape(BATCH, N_FIELDS * DIM)


# speedup vs baseline: 7.6346x; 1.0019x over previous
import jax
import jax.numpy as jnp
from jax import lax
from jax.experimental import pallas as pl
from jax.experimental.pallas import tpu as pltpu
from jax.experimental.pallas import tpu_sc as plsc

N_FIELDS = 26
VOCAB = 100000
DIM = 10
DIMP = 16
BATCH = 16384

NC = 2
NS = 16
L = 16
NW = NC * NS

ROWS_PER_W = BATCH * N_FIELDS // NW
G = 128
NG = ROWS_PER_W // G
VECS_PER_ROW = G // L

TAB_ROWS = N_FIELDS * VOCAB * DIM // DIMP
SLAB = N_FIELDS * VOCAB // DIMP

FIELD_STEP = (L % N_FIELDS) * VOCAB
FIELD_WRAP = N_FIELDS * VOCAB


def _body(idx_hbm, tab_hbm, out_hbm, idx_v, q_v, rows_v, out_v, gsem0,
          gsem1, gsem2, gsem3, osem0, osem1, osem2, osem3):
    wid = lax.axis_index("s") * NC + lax.axis_index("c")
    row0 = wid * ROWS_PER_W

    pltpu.sync_copy(idx_hbm.at[pl.ds(wid * NG, NG)], idx_v)

    foff0 = jax.lax.iota(jnp.int32, L) * VOCAB

    def idx_row(j, foff):
        for t in range(VECS_PER_ROW):
            sl = pl.ds(t * L, L)
            idx_v[j, sl] = idx_v[j, sl] + foff
            nxt = foff + FIELD_STEP
            foff = jnp.where(nxt >= FIELD_WRAP, nxt - FIELD_WRAP, nxt)
        return foff

    lax.fori_loop(0, NG, idx_row, foff0)

    gsems = (gsem0, gsem1, gsem2, gsem3)
    osems = (osem0, osem1, osem2, osem3)
    iota = jax.lax.iota(jnp.int32, L)

    def build_q(g, slot):
        for t in range(VECS_PER_ROW):
            sl = pl.ds(t * L, L)
            hi = jax.lax.shift_right_logical(idx_v[g, sl], 4)
            for k in range(DIM):
                q_v[slot, k, sl] = hi + (k * SLAB)

    def fire(g, slot):
        for k in range(DIM):
            pltpu.make_async_copy(
                tab_hbm.at[q_v.at[slot, k]],
                rows_v.at[slot, k],
                gsems[slot],
            ).start()

    def drain(slot):
        for k in range(DIM):
            pltpu.make_async_copy(
                tab_hbm.at[q_v.at[slot, k]],
                rows_v.at[slot, k],
                gsems[slot],
            ).wait()

    def reassemble(g, slot):
        for t in range(VECS_PER_ROW):
            sl = pl.ds(t * L, L)
            lo = jax.lax.bitwise_and(idx_v[g, sl], 15)
            jv = iota + (t * L)
            jd = jv * DIM
            for k in range(DIM):
                kv = jnp.full((L,), k, jnp.int32)
                vals = plsc.load_gather(rows_v.at[slot], [kv, jv, lo])
                plsc.store_scatter(out_v.at[slot], [jd + k], vals)

    def out_desc(g, slot):
        return pltpu.make_async_copy(
            out_v.at[slot],
            out_hbm.at[pl.ds((row0 + g * G) * DIM, G * DIM)],
            osems[slot])

    NSLOT = 4
    for s in range(NSLOT):
        build_q(s, s)
        fire(s, s)

    def step(j, _):
        g0 = NSLOT * j

        @pl.when(j > 0)
        def _():
            for s in range(NSLOT):
                out_desc(g0, s).wait()

        for s in range(NSLOT):
            g = g0 + s
            drain(s)
            reassemble(g, s)
            out_desc(g, s).start()
            build_q(g + NSLOT, s)
            fire(g + NSLOT, s)
        return 0

    lax.fori_loop(0, NG // NSLOT - 1, step, 0)
    for s in range(NSLOT):
        g = NG - NSLOT + s
        out_desc(g, s).wait()
        drain(s)
        reassemble(g, s)
        out_desc(g, s).start()
    for s in range(NSLOT):
        out_desc(NG - NSLOT + s, s).wait()


def _detile_body(tab_hbm, out_hbm, buf_v, ksem):
    wid = lax.axis_index("s") * NC + lax.axis_index("c")
    nlines = jnp.where(wid < 4, 9, 8)
    base = jnp.where(wid < 4, wid * 9, 36 + (wid - 4) * 8)

    def line(i, _):
        u = base + i
        k = u // N_FIELDS
        f = u % N_FIELDS
        pltpu.make_async_copy(tab_hbm.at[k, f], buf_v, ksem).start()
        pltpu.make_async_copy(tab_hbm.at[k, f], buf_v, ksem).wait()
        pltpu.sync_copy(
            buf_v,
            out_hbm.at[pl.ds(k * (N_FIELDS * VOCAB) + f * VOCAB, VOCAB)])
        return 0

    lax.fori_loop(0, nlines, line, 0)


@jax.jit
def _detile(tab_t):
    run = pl.kernel(
        _detile_body,
        out_type=jax.ShapeDtypeStruct((N_FIELDS * VOCAB * DIM,),
                                      jnp.float32),
        mesh=plsc.VectorSubcoreMesh(
            core_axis_name="c", subcore_axis_name="s",
            num_cores=NC, num_subcores=NS,
        ),
        scratch_types=[
            pltpu.VMEM((VOCAB,), jnp.float32),
            pltpu.SemaphoreType.DMA,
        ],
        compiler_params=pltpu.CompilerParams(use_tc_tiling_on_sc=True),
    )
    return run(tab_t)


@jax.jit
def _group_embed(idx_flat, tab_kmaj):
    run = pl.kernel(
        _body,
        out_type=jax.ShapeDtypeStruct((BATCH * N_FIELDS * DIM,),
                                      jnp.float32),
        mesh=plsc.VectorSubcoreMesh(
            core_axis_name="c", subcore_axis_name="s",
            num_cores=NC, num_subcores=NS,
        ),
        scratch_types=[
            pltpu.VMEM((NG, G), jnp.int32),
            pltpu.VMEM((4, DIM, G), jnp.int32),
            pltpu.VMEM((4, DIM, G, DIMP), jnp.float32),
            pltpu.VMEM((4, G * DIM), jnp.float32),
            pltpu.SemaphoreType.DMA,
            pltpu.SemaphoreType.DMA,
            pltpu.SemaphoreType.DMA,
            pltpu.SemaphoreType.DMA,
            pltpu.SemaphoreType.DMA,
            pltpu.SemaphoreType.DMA,
            pltpu.SemaphoreType.DMA,
            pltpu.SemaphoreType.DMA,
        ],
        compiler_params=pltpu.CompilerParams(
            use_tc_tiling_on_sc=False, needs_layout_passes=False),
    )
    return run(idx_flat, tab_kmaj)


def kernel(indices, tables):
    idx_flat = indices.reshape(BATCH * N_FIELDS // G, G)
    tab_kmaj = _detile(tables.transpose(2, 0, 1)).reshape(TAB_ROWS, DIMP)
    out = _group_embed(idx_flat, tab_kmaj)
    return out.reshape(BATCH, N_FIELDS * DIM)
